# baseline (speedup 1.0000x reference)
import functools

import jax
import jax.numpy as jnp
from jax import lax
from jax.experimental import pallas as pl
from jax.experimental.pallas import tpu as pltpu
from jax.experimental.pallas import tpu_sc as plsc

_VOCAB = 1000
_VPAD = 1024
_B = 1024
_L = 50
_LPAD = 56
_LP64 = 64
_N = _B * _L
_NC = 2
_NS = 16
_NW = _NC * _NS
_BPW = _B // _NW
_PER_W = _N // _NW


def _lse_body(emb_ref, out_ref):
    x = emb_ref[...]
    m = jnp.max(x, axis=1, keepdims=True)
    s = jnp.sum(jnp.exp(x - m), axis=1, keepdims=True)
    out_ref[...] = jnp.log(s) + m


def _compute_lse(embedding):
    out = pl.pallas_call(
        _lse_body,
        out_shape=jax.ShapeDtypeStruct((_VOCAB, 1), jnp.float32),
    )(embedding)
    return out.reshape(_VOCAB)


_sc_mesh = plsc.VectorSubcoreMesh(core_axis_name="c", subcore_axis_name="s")


@functools.partial(
    pl.kernel,
    out_type=(
        jax.ShapeDtypeStruct((_B, _LPAD, _VPAD), jnp.float32),
        jax.ShapeDtypeStruct((_N,), jnp.float32),
    ),
    mesh=_sc_mesh,
    compiler_params=pltpu.CompilerParams(
        use_tc_tiling_on_sc=False, needs_layout_passes=False),
    scratch_types=[
        pltpu.VMEM((_LPAD, _VPAD), jnp.float32),
        pltpu.VMEM((_LPAD, _VPAD), jnp.float32),
        pltpu.VMEM((_BPW * _LP64,), jnp.int32),
        pltpu.VMEM((_BPW * _LP64,), jnp.int32),
        pltpu.VMEM((_VOCAB,), jnp.float32),
        pltpu.VMEM((_PER_W + _LP64,), jnp.float32),
        pltpu.SemaphoreType.DMA,
        pltpu.SemaphoreType.DMA,
        pltpu.SemaphoreType.DMA,
        pltpu.SemaphoreType.DMA,
    ],
)
def _sc_main(emb_pad_hbm, idx64_hbm, tgt64_hbm, lse_hbm, out_hbm, loss_hbm,
             rows0, rows1, idx_v, tgt_v, lse_v, loss_all,
             sem_g0, sem_g1, sem_w0, sem_w1):
    wid = lax.axis_index("s") * _NC + lax.axis_index("c")
    wb = wid * _BPW
    base_w = wid * _PER_W
    pltpu.sync_copy(idx64_hbm.at[pl.ds(wb * _LP64, _BPW * _LP64)], idx_v)
    pltpu.sync_copy(tgt64_hbm.at[pl.ds(wb * _LP64, _BPW * _LP64)], tgt_v)
    pltpu.sync_copy(lse_hbm, lse_v)

    def issue_gather(c, buf, sem):
        pltpu.async_copy(
            emb_pad_hbm.at[idx_v.at[pl.ds(c * _LP64, _LPAD)]], buf, sem)

    def wait_gather(buf, sem):
        pltpu.make_async_copy(
            emb_pad_hbm.at[idx_v.at[pl.ds(0, _LPAD)]], buf, sem).wait()

    def issue_write(c, buf, sem):
        pltpu.async_copy(buf, out_hbm.at[wb + c], sem)

    def wait_write(buf, sem):
        pltpu.make_async_copy(buf, out_hbm.at[wb], sem).wait()

    def compute_loss(c, buf):
        coff = pl.multiple_of(c * _LP64, _LP64)
        for j in range(_LP64 // 16):
            n16 = lax.iota(jnp.int32, 16) + (j * 16)
            sl = pl.ds(coff + j * 16, 16)
            idx16 = idx_v[sl]
            tgt16 = tgt_v[sl]
            tl16 = plsc.load_gather(buf, [n16, tgt16])
            lse16 = plsc.load_gather(lse_v, [idx16])
            plsc.store_scatter(loss_all, [c * _L + n16], lse16 - tl16)

    issue_gather(0, rows0, sem_g0)
    issue_gather(1, rows1, sem_g1)

    def step(i, carry):
        c0 = i * 2
        wait_gather(rows0, sem_g0)
        issue_write(c0, rows0, sem_w0)
        compute_loss(c0, rows0)
        wait_gather(rows1, sem_g1)
        issue_write(c0 + 1, rows1, sem_w1)
        compute_loss(c0 + 1, rows1)
        wait_write(rows0, sem_w0)
        issue_gather(c0 + 2, rows0, sem_g0)
        wait_write(rows1, sem_w1)
        issue_gather(c0 + 3, rows1, sem_g1)
        return carry

    lax.fori_loop(0, _BPW // 2 - 1, step, 0)

    wait_gather(rows0, sem_g0)
    issue_write(_BPW - 2, rows0, sem_w0)
    compute_loss(_BPW - 2, rows0)
    wait_gather(rows1, sem_g1)
    issue_write(_BPW - 1, rows1, sem_w1)
    compute_loss(_BPW - 1, rows1)
    wait_write(rows0, sem_w0)
    wait_write(rows1, sem_w1)
    pltpu.sync_copy(loss_all.at[pl.ds(0, _PER_W)],
                    loss_hbm.at[pl.ds(base_w, _PER_W)])


def kernel(indices, targets, embedding):
    emb_pad = jnp.pad(embedding, ((0, 0), (0, _VPAD - _VOCAB)))
    idx64 = jnp.pad(indices, ((0, 0), (0, _LP64 - _L))).reshape(_B * _LP64)
    tgt64 = jnp.pad(targets, ((0, 0), (0, _LP64 - _L))).reshape(_B * _LP64)
    lse = _compute_lse(embedding)
    out_p, loss = _sc_main(emb_pad, idx64, tgt64, lse)
    return out_p[:, :_L, :_VOCAB], loss


# speedup vs baseline: 1.4720x; 1.4720x over previous
import functools

import jax
import jax.numpy as jnp
from jax import lax
from jax.experimental import pallas as pl
from jax.experimental.pallas import tpu as pltpu
from jax.experimental.pallas import tpu_sc as plsc

_VOCAB = 1000
_B = 1024
_L = 50
_LP = 64
_N = _B * _L
_NC = 2
_NS = 16
_NW = _NC * _NS
_BPW = _B // _NW
_PER_W = _N // _NW


def _lse_body(emb_ref, out_ref):
    x = emb_ref[...]
    m = jnp.max(x, axis=1, keepdims=True)
    s = jnp.sum(jnp.exp(x - m), axis=1, keepdims=True)
    out_ref[...] = jnp.log(s) + m


def _compute_lse(embedding):
    out = pl.pallas_call(
        _lse_body,
        out_shape=jax.ShapeDtypeStruct((_VOCAB, 1), jnp.float32),
    )(embedding)
    return out.reshape(_VOCAB)


_sc_mesh = plsc.VectorSubcoreMesh(core_axis_name="c", subcore_axis_name="s")


@functools.partial(
    pl.kernel,
    out_type=(
        jax.ShapeDtypeStruct((_B, _L, _VOCAB), jnp.float32),
        jax.ShapeDtypeStruct((_N,), jnp.float32),
    ),
    mesh=_sc_mesh,
    compiler_params=pltpu.CompilerParams(
        use_tc_tiling_on_sc=False, needs_layout_passes=False),
    scratch_types=[
        pltpu.VMEM((_BPW, _L), jnp.int32),
        pltpu.VMEM((_BPW, _LP), jnp.int32),
        pltpu.VMEM((_BPW, _LP), jnp.int32),
        pltpu.VMEM((_L, _VOCAB), jnp.float32),
        pltpu.VMEM((_L, _VOCAB), jnp.float32),
        pltpu.VMEM((_VOCAB,), jnp.float32),
        pltpu.VMEM((_PER_W + _LP,), jnp.float32),
        pltpu.SemaphoreType.DMA,
        pltpu.SemaphoreType.DMA,
        pltpu.SemaphoreType.DMA,
        pltpu.SemaphoreType.DMA,
    ],
)
def _sc_main(emb_hbm, idx2d_hbm, idxp_hbm, tgtp_hbm, lse_hbm,
             out_hbm, loss_hbm,
             idx2d_v, idxp_v, tgtp_v, rows0, rows1, lse_v, loss_all,
             sem_g0, sem_g1, sem_w0, sem_w1):
    wid = lax.axis_index("s") * _NC + lax.axis_index("c")
    wb = wid * _BPW
    base_w = wid * _PER_W
    pltpu.sync_copy(idx2d_hbm.at[pl.ds(wb, _BPW)], idx2d_v)
    pltpu.sync_copy(idxp_hbm.at[pl.ds(wb, _BPW)], idxp_v)
    pltpu.sync_copy(tgtp_hbm.at[pl.ds(wb, _BPW)], tgtp_v)
    pltpu.sync_copy(lse_hbm, lse_v)

    def issue_gather(c, buf, sem):
        pltpu.async_copy(emb_hbm.at[idx2d_v.at[c]], buf, sem)

    def wait_gather(buf, sem):
        pltpu.make_async_copy(emb_hbm.at[idx2d_v.at[0]], buf, sem).wait()

    def issue_write(c, buf, sem):
        pltpu.async_copy(buf, out_hbm.at[wb + c], sem)

    def wait_write(buf, sem):
        pltpu.make_async_copy(buf, out_hbm.at[wb], sem).wait()

    def compute_loss(c, buf):
        for j in range(_LP // 16):
            n16 = lax.iota(jnp.int32, 16) + (j * 16)
            sl = pl.ds(j * 16, 16)
            idx16 = idxp_v[c, sl]
            tgt16 = tgtp_v[c, sl]
            tl16 = plsc.load_gather(buf, [n16, tgt16])
            lse16 = plsc.load_gather(lse_v, [idx16])
            plsc.store_scatter(loss_all, [c * _L + n16], lse16 - tl16)

    issue_gather(0, rows0, sem_g0)
    issue_gather(1, rows1, sem_g1)

    def step(i, carry):
        c0 = i * 2
        wait_gather(rows0, sem_g0)
        issue_write(c0, rows0, sem_w0)
        compute_loss(c0, rows0)
        wait_gather(rows1, sem_g1)
        issue_write(c0 + 1, rows1, sem_w1)
        compute_loss(c0 + 1, rows1)
        wait_write(rows0, sem_w0)
        issue_gather(c0 + 2, rows0, sem_g0)
        wait_write(rows1, sem_w1)
        issue_gather(c0 + 3, rows1, sem_g1)
        return carry

    lax.fori_loop(0, _BPW // 2 - 1, step, 0)

    wait_gather(rows0, sem_g0)
    issue_write(_BPW - 2, rows0, sem_w0)
    compute_loss(_BPW - 2, rows0)
    wait_gather(rows1, sem_g1)
    issue_write(_BPW - 1, rows1, sem_w1)
    compute_loss(_BPW - 1, rows1)
    wait_write(rows0, sem_w0)
    wait_write(rows1, sem_w1)
    pltpu.sync_copy(loss_all.at[pl.ds(0, _PER_W)],
                    loss_hbm.at[pl.ds(base_w, _PER_W)])


def kernel(indices, targets, embedding):
    idx_pad = jnp.pad(indices, ((0, 0), (0, _LP - _L)))
    tgt_pad = jnp.pad(targets, ((0, 0), (0, _LP - _L)))
    lse = _compute_lse(embedding)
    logits, loss = _sc_main(embedding, indices, idx_pad, tgt_pad, lse)
    return logits, loss
